# baseline (speedup 1.0000x reference)
import functools

import numpy as np

import jax
import jax.numpy as jnp
from jax import lax
from jax.experimental import pallas as pl
from jax.experimental.pallas import tpu as pltpu
from jax.experimental.pallas import tpu_sc as plsc

N = 10000
E = 320000
IN_DIM = 128
HIDDEN = 128
OUT = 64

NC = 2
NS = 16
K = 128
CH = 84
NWIN = 2
NCHW = 42
EPC = NS * CH * K
N_PAD = 10112
RPT = N_PAD // NS

_MESH = plsc.VectorSubcoreMesh(core_axis_name="c", subcore_axis_name="s")

_ROW_CHUNKS = [(i * K, K) for i in range(RPT // K)]
if RPT % K:
    _ROW_CHUNKS.append((RPT - RPT % K, RPT % K))

_PERM_LOCAL = np.array([(q // 2) if q % 2 == 0 else 16 + (q // 2)
                        for q in range(32)])
COL_PERM = np.concatenate([b * 32 + _PERM_LOCAL for b in range(4)])


@functools.partial(
    pl.kernel,
    out_type=jax.ShapeDtypeStruct((NC * N_PAD,), jnp.float32),
    mesh=_MESH,
    scratch_types=[
        pltpu.VMEM_SHARED((N_PAD,), jnp.float32),
        pltpu.VMEM((NCHW, K // 128, 128), jnp.int32),
        pltpu.VMEM((128,), jnp.float32),
        pltpu.VMEM((RPT,), jnp.float32),
    ],
)
def _degree_kernel(dst_hbm, init_hbm, deg_out, deg_sh, idx_v, ones_v, stage_v):
    c = lax.axis_index("c")
    s = lax.axis_index("s")
    pltpu.sync_copy(init_hbm.at[pl.ds(c * N_PAD + s * RPT, RPT)], stage_v)
    pltpu.sync_copy(stage_v, deg_sh.at[pl.ds(s * RPT, RPT)])
    for i in range(128 // 16):
        ones_v[pl.ds(i * 16, 16)] = jnp.ones((16,), jnp.float32)
    plsc.subcore_barrier()

    def body(j, carry):
        for i in range(K // 128):
            pltpu.sync_copy(ones_v, deg_sh.at[idx_v.at[j, i]], add=True)
        return carry

    for h in range(NWIN):
        pltpu.sync_copy(dst_hbm.at[c, s, h], idx_v)
        lax.fori_loop(0, NCHW, body, 0)
    plsc.subcore_barrier()
    pltpu.sync_copy(deg_sh.at[pl.ds(s * RPT, RPT)], stage_v)
    pltpu.sync_copy(stage_v, deg_out.at[pl.ds(c * N_PAD + s * RPT, RPT)])


@functools.partial(
    pl.kernel,
    out_type=jax.ShapeDtypeStruct((NC, N_PAD, HIDDEN), jnp.float32),
    mesh=_MESH,
    scratch_types=[
        pltpu.VMEM_SHARED((N_PAD, HIDDEN), jnp.float32),
        pltpu.VMEM((NCHW, K), jnp.int32),
        pltpu.VMEM((NCHW, K), jnp.int32),
        pltpu.VMEM((K, HIDDEN), jnp.float32),
    ],
)
def _propagate(p_hbm, src_hbm, dst_hbm, out_hbm, acc_sh, src_v, dst_v,
               rows_f):
    c = lax.axis_index("c")
    s = lax.axis_index("s")

    def zrow(i, carry):
        for jb in range(HIDDEN // 16):
            rows_f[i, pl.ds(jb * 16, 16)] = jnp.zeros((16,), jnp.float32)
        return carry

    lax.fori_loop(0, K, zrow, 0)
    for off, sz in _ROW_CHUNKS:
        pltpu.sync_copy(rows_f.at[pl.ds(0, sz)],
                        acc_sh.at[pl.ds(s * RPT + off, sz)])
    plsc.subcore_barrier()

    def body(j, carry):
        pltpu.sync_copy(p_hbm.at[src_v.at[j]], rows_f)
        pltpu.sync_copy(rows_f, acc_sh.at[dst_v.at[j]], add=True)
        return carry

    for h in range(NWIN):
        pltpu.sync_copy(src_hbm.at[c, s, h], src_v)
        pltpu.sync_copy(dst_hbm.at[c, s, h], dst_v)
        lax.fori_loop(0, NCHW, body, 0)

    plsc.subcore_barrier()
    for off, sz in _ROW_CHUNKS:
        pltpu.sync_copy(acc_sh.at[pl.ds(s * RPT + off, sz)],
                        rows_f.at[pl.ds(0, sz)])
        pltpu.sync_copy(rows_f.at[pl.ds(0, sz)],
                        out_hbm.at[c, pl.ds(s * RPT + off, sz)])


def _entry_body(degp_ref, x_ref, w_ref, dinv_ref, p_ref):
    deg = degp_ref[0, :] + degp_ref[1, :]
    dinv = lax.rsqrt(deg)
    dinv_ref[...] = dinv[:, None]
    p_ref[...] = dinv[:, None] * jnp.dot(
        x_ref[...].astype(jnp.bfloat16), w_ref[...],
        preferred_element_type=jnp.float32)


_tc_entry = pl.pallas_call(
    _entry_body,
    out_shape=(
        jax.ShapeDtypeStruct((N_PAD, 1), jnp.float32),
        jax.ShapeDtypeStruct((N_PAD, IN_DIM), jnp.float32),
    ),
)


def _mid_body(s_ref, dinv_ref, b_ref, w_ref, o_ref):
    h = jnp.maximum(dinv_ref[...] * (s_ref[0] + s_ref[1]) + b_ref[...], 0.0)
    o_ref[...] = dinv_ref[...] * jnp.dot(
        h.astype(jnp.bfloat16), w_ref[...],
        preferred_element_type=jnp.float32)


_tc_mid = pl.pallas_call(
    _mid_body,
    out_shape=jax.ShapeDtypeStruct((N_PAD, HIDDEN), jnp.float32),
)


def _final_body(s_ref, dinv_ref, b_ref, o_ref):
    o_ref[...] = (dinv_ref[...] * (s_ref[0, :, :OUT] + s_ref[1, :, :OUT])
                  + b_ref[...])


_tc_final = pl.pallas_call(
    _final_body,
    out_shape=jax.ShapeDtypeStruct((N_PAD, OUT), jnp.float32),
)


def kernel(x, ei, W1, b1, W2, b2, W3, b3):
    half = E // 2
    nh = N // 2
    loop = jnp.arange(N, dtype=jnp.int32)
    njunk = EPC - half - nh
    junk_src = jnp.full((njunk,), N, dtype=jnp.int32)
    junk_dst = N + jnp.arange(njunk, dtype=jnp.int32) % (N_PAD - N)

    def tiles(v, junk):
        v = v.astype(jnp.int32)
        sc0 = jnp.concatenate([v[:half], loop[:nh], junk])
        sc1 = jnp.concatenate([v[half:], loop[nh:], junk])
        return jnp.stack([sc0, sc1]).reshape(NC, NS, NWIN, NCHW, K)

    src_t = tiles(ei[0], junk_src)
    dst_t = tiles(ei[1], junk_dst)
    xp = jnp.pad(x, ((0, N_PAD - N), (0, 0)))
    init = jnp.zeros((NC * N_PAD,), jnp.float32)
    w1p = W1.astype(jnp.bfloat16)
    w2p = W2.astype(jnp.bfloat16)
    w3p = jnp.pad(W3, ((0, 0), (0, HIDDEN - OUT))).astype(jnp.bfloat16)

    degp = _degree_kernel(
        dst_t.reshape(NC, NS, NWIN, NCHW, K // 128, 128), init
    ).reshape(NC, N_PAD)
    dinv, p1 = _tc_entry(degp, xp, w1p)
    s1 = _propagate(p1, src_t, dst_t)
    p2 = _tc_mid(s1, dinv, b1.reshape(1, -1), w2p)
    s2 = _propagate(p2, src_t, dst_t)
    p3 = _tc_mid(s2, dinv, b2.reshape(1, -1), w3p)
    s3 = _propagate(p3, src_t, dst_t)
    out = _tc_final(s3, dinv, b3.reshape(1, -1))
    return out[:N]


# speedup vs baseline: 2.5006x; 2.5006x over previous
import functools

import numpy as np

import jax
import jax.numpy as jnp
from jax import lax
from jax.experimental import pallas as pl
from jax.experimental.pallas import tpu as pltpu
from jax.experimental.pallas import tpu_sc as plsc

N = 10000
E = 320000
IN_DIM = 128
HIDDEN = 128
OUT = 64

NC = 2
NS = 16
K = 128
CH = 81
NWIN = 3
NCHW = 27
EPC = NS * CH * K
N_PAD = 10112
RPT = N_PAD // NS

_MESH = plsc.VectorSubcoreMesh(core_axis_name="c", subcore_axis_name="s")

_ROW_CHUNKS = [(i * K, K) for i in range(RPT // K)]
if RPT % K:
    _ROW_CHUNKS.append((RPT - RPT % K, RPT % K))

_PERM_LOCAL = np.array([(q // 2) if q % 2 == 0 else 16 + (q // 2)
                        for q in range(32)])
COL_PERM = np.concatenate([b * 32 + _PERM_LOCAL for b in range(4)])


@functools.partial(
    pl.kernel,
    out_type=jax.ShapeDtypeStruct((NC * N_PAD,), jnp.float32),
    mesh=_MESH,
    scratch_types=[
        pltpu.VMEM_SHARED((N_PAD,), jnp.float32),
        pltpu.VMEM((NCHW, K // 128, 128), jnp.int32),
        pltpu.VMEM((128,), jnp.float32),
        pltpu.VMEM((RPT,), jnp.float32),
    ],
)
def _degree_kernel(dst_hbm, init_hbm, deg_out, deg_sh, idx_v, ones_v, stage_v):
    c = lax.axis_index("c")
    s = lax.axis_index("s")
    pltpu.sync_copy(init_hbm.at[pl.ds(c * N_PAD + s * RPT, RPT)], stage_v)
    pltpu.sync_copy(stage_v, deg_sh.at[pl.ds(s * RPT, RPT)])
    for i in range(128 // 16):
        ones_v[pl.ds(i * 16, 16)] = jnp.ones((16,), jnp.float32)
    plsc.subcore_barrier()

    def body(j, carry):
        for i in range(K // 128):
            pltpu.sync_copy(ones_v, deg_sh.at[idx_v.at[j, i]], add=True)
        return carry

    for h in range(NWIN):
        pltpu.sync_copy(dst_hbm.at[c, s, h], idx_v)
        lax.fori_loop(0, NCHW, body, 0)
    plsc.subcore_barrier()
    pltpu.sync_copy(deg_sh.at[pl.ds(s * RPT, RPT)], stage_v)
    pltpu.sync_copy(stage_v, deg_out.at[pl.ds(c * N_PAD + s * RPT, RPT)])


@functools.partial(
    pl.kernel,
    out_type=jax.ShapeDtypeStruct((NC, N_PAD, HIDDEN), jnp.float32),
    mesh=_MESH,
    scratch_types=[
        pltpu.VMEM_SHARED((N_PAD, HIDDEN), jnp.float32),
        pltpu.VMEM((NCHW, K), jnp.int32),
        pltpu.VMEM((NCHW, K), jnp.int32),
        pltpu.VMEM((K, HIDDEN), jnp.float32),
    ],
)
def _propagate(p_hbm, src_hbm, dst_hbm, out_hbm, acc_sh, src_v, dst_v,
               rows_f):
    c = lax.axis_index("c")
    s = lax.axis_index("s")

    def zrow(i, carry):
        for jb in range(HIDDEN // 16):
            rows_f[i, pl.ds(jb * 16, 16)] = jnp.zeros((16,), jnp.float32)
        return carry

    lax.fori_loop(0, K, zrow, 0)
    for off, sz in _ROW_CHUNKS:
        pltpu.sync_copy(rows_f.at[pl.ds(0, sz)],
                        acc_sh.at[pl.ds(s * RPT + off, sz)])
    plsc.subcore_barrier()

    def body(j, carry):
        pltpu.sync_copy(p_hbm.at[src_v.at[j]], rows_f)
        pltpu.sync_copy(rows_f, acc_sh.at[dst_v.at[j]], add=True)
        return carry

    for h in range(NWIN):
        pltpu.sync_copy(src_hbm.at[c, s, h], src_v)
        pltpu.sync_copy(dst_hbm.at[c, s, h], dst_v)
        lax.fori_loop(0, NCHW, body, 0)

    plsc.subcore_barrier()
    for off, sz in _ROW_CHUNKS:
        pltpu.sync_copy(acc_sh.at[pl.ds(s * RPT + off, sz)],
                        rows_f.at[pl.ds(0, sz)])
        pltpu.sync_copy(rows_f.at[pl.ds(0, sz)],
                        out_hbm.at[c, pl.ds(s * RPT + off, sz)])


def _entry_body(degp_ref, x_ref, w_ref, dinv_ref, p_ref):
    deg = degp_ref[0, :] + degp_ref[1, :]
    dinv = lax.rsqrt(deg)
    dinv_ref[...] = dinv[:, None]
    p_ref[...] = dinv[:, None] * jnp.dot(
        x_ref[...].astype(jnp.bfloat16), w_ref[...],
        preferred_element_type=jnp.float32)


_tc_entry = pl.pallas_call(
    _entry_body,
    out_shape=(
        jax.ShapeDtypeStruct((N_PAD, 1), jnp.float32),
        jax.ShapeDtypeStruct((N_PAD, IN_DIM), jnp.float32),
    ),
)


def _mid_body(s_ref, dinv_ref, b_ref, w_ref, o_ref):
    h = jnp.maximum(dinv_ref[...] * (s_ref[0] + s_ref[1]) + b_ref[...], 0.0)
    o_ref[...] = dinv_ref[...] * jnp.dot(
        h.astype(jnp.bfloat16), w_ref[...],
        preferred_element_type=jnp.float32)


_tc_mid = pl.pallas_call(
    _mid_body,
    out_shape=jax.ShapeDtypeStruct((N_PAD, HIDDEN), jnp.float32),
)


def _final_body(s_ref, dinv_ref, b_ref, o_ref):
    o_ref[...] = (dinv_ref[...] * (s_ref[0, :, :OUT] + s_ref[1, :, :OUT])
                  + b_ref[...])


_tc_final = pl.pallas_call(
    _final_body,
    out_shape=jax.ShapeDtypeStruct((N_PAD, OUT), jnp.float32),
)


def kernel(x, ei, W1, b1, W2, b2, W3, b3):
    half = E // 2
    nh = N // 2
    loop = jnp.arange(N, dtype=jnp.int32)
    njunk = EPC - half - nh
    junk_src = jnp.full((njunk,), N, dtype=jnp.int32)
    junk_dst = jnp.arange(njunk, dtype=jnp.int32) % N
    junk_deg = N + jnp.arange(njunk, dtype=jnp.int32) % (N_PAD - N)

    def tiles(v, junk):
        v = v.astype(jnp.int32)
        sc0 = jnp.concatenate([v[:half], loop[:nh], junk])
        sc1 = jnp.concatenate([v[half:], loop[nh:], junk])
        return jnp.stack([sc0, sc1]).reshape(NC, NS, NWIN, NCHW, K)

    src_t = tiles(ei[0], junk_src)
    dst_t = tiles(ei[1], junk_dst)
    deg_t = tiles(ei[1], junk_deg)
    xp = jnp.pad(x, ((0, N_PAD - N), (0, 0)))
    init = jnp.zeros((NC * N_PAD,), jnp.float32)
    w1p = W1.astype(jnp.bfloat16)
    w2p = W2.astype(jnp.bfloat16)
    w3p = jnp.pad(W3, ((0, 0), (0, HIDDEN - OUT))).astype(jnp.bfloat16)

    degp = _degree_kernel(
        deg_t.reshape(NC, NS, NWIN, NCHW, K // 128, 128), init
    ).reshape(NC, N_PAD)
    dinv, p1 = _tc_entry(degp, xp, w1p)
    s1 = _propagate(p1, src_t, dst_t)
    p2 = _tc_mid(s1, dinv, b1.reshape(1, -1), w2p)
    s2 = _propagate(p2, src_t, dst_t)
    p3 = _tc_mid(s2, dinv, b2.reshape(1, -1), w3p)
    s3 = _propagate(p3, src_t, dst_t)
    out = _tc_final(s3, dinv, b3.reshape(1, -1))
    return out[:N]


# speedup vs baseline: 2.5061x; 1.0022x over previous
import functools

import numpy as np

import jax
import jax.numpy as jnp
from jax import lax
from jax.experimental import pallas as pl
from jax.experimental.pallas import tpu as pltpu
from jax.experimental.pallas import tpu_sc as plsc

N = 10000
E = 320000
IN_DIM = 128
HIDDEN = 128
OUT = 64

NC = 2
NS = 16
K = 128
CH = 81
NWIN = 3
NCHW = 27
EPC = NS * CH * K
N_PAD = 10112
RPT = N_PAD // NS

_MESH = plsc.VectorSubcoreMesh(core_axis_name="c", subcore_axis_name="s")

_ROW_CHUNKS = [(i * K, K) for i in range(RPT // K)]
if RPT % K:
    _ROW_CHUNKS.append((RPT - RPT % K, RPT % K))

_PERM_LOCAL = np.array([(q // 2) if q % 2 == 0 else 16 + (q // 2)
                        for q in range(32)])
COL_PERM = np.concatenate([b * 32 + _PERM_LOCAL for b in range(4)])


@functools.partial(
    pl.kernel,
    out_type=jax.ShapeDtypeStruct((NC * N_PAD,), jnp.float32),
    mesh=_MESH,
    scratch_types=[
        pltpu.VMEM_SHARED((N_PAD,), jnp.float32),
        pltpu.VMEM((NCHW, K // 128, 128), jnp.int32),
        pltpu.VMEM((128,), jnp.float32),
        pltpu.VMEM((RPT,), jnp.float32),
    ],
)
def _degree_kernel(dst_hbm, init_hbm, deg_out, deg_sh, idx_v, ones_v, stage_v):
    c = lax.axis_index("c")
    s = lax.axis_index("s")
    pltpu.sync_copy(init_hbm.at[pl.ds(c * N_PAD + s * RPT, RPT)], stage_v)
    pltpu.sync_copy(stage_v, deg_sh.at[pl.ds(s * RPT, RPT)])
    for i in range(128 // 16):
        ones_v[pl.ds(i * 16, 16)] = jnp.ones((16,), jnp.float32)
    plsc.subcore_barrier()

    def body(j, carry):
        for i in range(K // 128):
            pltpu.sync_copy(ones_v, deg_sh.at[idx_v.at[j, i]], add=True)
        return carry

    for h in range(NWIN):
        pltpu.sync_copy(dst_hbm.at[c, s, h], idx_v)
        lax.fori_loop(0, NCHW, body, 0)
    plsc.subcore_barrier()
    pltpu.sync_copy(deg_sh.at[pl.ds(s * RPT, RPT)], stage_v)
    pltpu.sync_copy(stage_v, deg_out.at[pl.ds(c * N_PAD + s * RPT, RPT)])


@functools.partial(
    pl.kernel,
    out_type=jax.ShapeDtypeStruct((NC, N_PAD, HIDDEN), jnp.float32),
    mesh=_MESH,
    scratch_types=[
        pltpu.VMEM_SHARED((N_PAD, HIDDEN), jnp.float32),
        pltpu.VMEM((NCHW, K), jnp.int32),
        pltpu.VMEM((NCHW, K), jnp.int32),
        pltpu.VMEM((K, HIDDEN), jnp.float32),
    ],
)
def _propagate(p_hbm, src_hbm, dst_hbm, out_hbm, acc_sh, src_v, dst_v,
               rows_f):
    c = lax.axis_index("c")
    s = lax.axis_index("s")

    def zrow(i, carry):
        for jb in range(HIDDEN // 16):
            rows_f[i, pl.ds(jb * 16, 16)] = jnp.zeros((16,), jnp.float32)
        return carry

    lax.fori_loop(0, K, zrow, 0)
    for off, sz in _ROW_CHUNKS:
        pltpu.sync_copy(rows_f.at[pl.ds(0, sz)],
                        acc_sh.at[pl.ds(s * RPT + off, sz)])
    plsc.subcore_barrier()

    def body(j, carry):
        pltpu.sync_copy(p_hbm.at[src_v.at[j]], rows_f)
        pltpu.sync_copy(rows_f, acc_sh.at[dst_v.at[j]], add=True)
        return carry

    for h in range(NWIN):
        pltpu.sync_copy(src_hbm.at[c, s, h], src_v)
        pltpu.sync_copy(dst_hbm.at[c, s, h], dst_v)
        lax.fori_loop(0, NCHW, body, 0)

    plsc.subcore_barrier()
    for off, sz in _ROW_CHUNKS:
        pltpu.sync_copy(acc_sh.at[pl.ds(s * RPT + off, sz)],
                        rows_f.at[pl.ds(0, sz)])
        pltpu.sync_copy(rows_f.at[pl.ds(0, sz)],
                        out_hbm.at[c, pl.ds(s * RPT + off, sz)])


def _entry_body(degp_ref, x_ref, w_ref, dinv_ref, p_ref):
    deg = degp_ref[0, :] + degp_ref[1, :]
    dinv = lax.rsqrt(deg)
    dinv_ref[...] = dinv[:, None]
    p_ref[...] = dinv[:, None] * jnp.dot(
        x_ref[...], w_ref[...], preferred_element_type=jnp.float32)


_tc_entry = pl.pallas_call(
    _entry_body,
    out_shape=(
        jax.ShapeDtypeStruct((N_PAD, 1), jnp.float32),
        jax.ShapeDtypeStruct((N_PAD, IN_DIM), jnp.float32),
    ),
)


def _mid_body(s_ref, dinv_ref, b_ref, w_ref, o_ref):
    h = jnp.maximum(dinv_ref[...] * (s_ref[0] + s_ref[1]) + b_ref[...], 0.0)
    o_ref[...] = dinv_ref[...] * jnp.dot(
        h, w_ref[...], preferred_element_type=jnp.float32)


_tc_mid = pl.pallas_call(
    _mid_body,
    out_shape=jax.ShapeDtypeStruct((N_PAD, HIDDEN), jnp.float32),
)


def _final_body(s_ref, dinv_ref, b_ref, o_ref):
    o_ref[...] = (dinv_ref[...] * (s_ref[0, :, :OUT] + s_ref[1, :, :OUT])
                  + b_ref[...])


_tc_final = pl.pallas_call(
    _final_body,
    out_shape=jax.ShapeDtypeStruct((N_PAD, OUT), jnp.float32),
)


def kernel(x, ei, W1, b1, W2, b2, W3, b3):
    half = E // 2
    nh = N // 2
    loop = jnp.arange(N, dtype=jnp.int32)
    njunk = EPC - half - nh
    junk_src = jnp.full((njunk,), N, dtype=jnp.int32)
    junk_dst = jnp.arange(njunk, dtype=jnp.int32) % N
    junk_deg = N + jnp.arange(njunk, dtype=jnp.int32) % (N_PAD - N)

    def tiles(v, junk):
        v = v.astype(jnp.int32)
        sc0 = jnp.concatenate([v[:half], loop[:nh], junk])
        sc1 = jnp.concatenate([v[half:], loop[nh:], junk])
        return jnp.stack([sc0, sc1]).reshape(NC, NS, NWIN, NCHW, K)

    src_t = tiles(ei[0], junk_src)
    dst_t = tiles(ei[1], junk_dst)
    deg_t = tiles(ei[1], junk_deg)
    xp = jnp.pad(x, ((0, N_PAD - N), (0, 0)))
    init = jnp.zeros((NC * N_PAD,), jnp.float32)
    w1p = W1
    w2p = W2
    w3p = jnp.pad(W3, ((0, 0), (0, HIDDEN - OUT)))

    degp = _degree_kernel(
        deg_t.reshape(NC, NS, NWIN, NCHW, K // 128, 128), init
    ).reshape(NC, N_PAD)
    dinv, p1 = _tc_entry(degp, xp, w1p)
    s1 = _propagate(p1, src_t, dst_t)
    p2 = _tc_mid(s1, dinv, b1.reshape(1, -1), w2p)
    s2 = _propagate(p2, src_t, dst_t)
    p3 = _tc_mid(s2, dinv, b2.reshape(1, -1), w3p)
    s3 = _propagate(p3, src_t, dst_t)
    out = _tc_final(s3, dinv, b3.reshape(1, -1))
    return out[:N]


# speedup vs baseline: 2.5539x; 1.0191x over previous
import functools

import numpy as np

import jax
import jax.numpy as jnp
from jax import lax
from jax.experimental import pallas as pl
from jax.experimental.pallas import tpu as pltpu
from jax.experimental.pallas import tpu_sc as plsc

N = 10000
E = 320000
IN_DIM = 128
HIDDEN = 128
OUT = 64

NC = 2
NS = 16
K = 128
CH = 81
NWIN = 3
NCHW = 27
EPC = NS * CH * K
N_PAD = 10112
RPT = N_PAD // NS

_MESH = plsc.VectorSubcoreMesh(core_axis_name="c", subcore_axis_name="s")

_ROW_CHUNKS = [(i * K, K) for i in range(RPT // K)]
if RPT % K:
    _ROW_CHUNKS.append((RPT - RPT % K, RPT % K))

_PERM_LOCAL = np.array([(q // 2) if q % 2 == 0 else 16 + (q // 2)
                        for q in range(32)])
COL_PERM = np.concatenate([b * 32 + _PERM_LOCAL for b in range(4)])


@functools.partial(
    pl.kernel,
    out_type=jax.ShapeDtypeStruct((NC * N_PAD,), jnp.float32),
    mesh=_MESH,
    scratch_types=[
        pltpu.VMEM_SHARED((N_PAD,), jnp.float32),
        pltpu.VMEM((NCHW, K), jnp.int32),
        pltpu.VMEM((K,), jnp.float32),
        pltpu.VMEM((RPT,), jnp.float32),
    ],
)
def _degree_kernel(dst_hbm, init_hbm, deg_out, deg_sh, idx_v, ones_v, stage_v):
    c = lax.axis_index("c")
    s = lax.axis_index("s")
    pltpu.sync_copy(init_hbm.at[pl.ds(c * N_PAD + s * RPT, RPT)], stage_v)
    pltpu.sync_copy(stage_v, deg_sh.at[pl.ds(s * RPT, RPT)])
    for i in range(K // 16):
        ones_v[pl.ds(i * 16, 16)] = jnp.ones((16,), jnp.float32)
    plsc.subcore_barrier()

    def body(j, carry):
        pltpu.sync_copy(ones_v, deg_sh.at[idx_v.at[j]], add=True)
        return carry

    for h in range(NWIN):
        pltpu.sync_copy(dst_hbm.at[c, s, h], idx_v)
        lax.fori_loop(0, NCHW, body, 0)
    plsc.subcore_barrier()
    pltpu.sync_copy(deg_sh.at[pl.ds(s * RPT, RPT)], stage_v)
    pltpu.sync_copy(stage_v, deg_out.at[pl.ds(c * N_PAD + s * RPT, RPT)])


@functools.partial(
    pl.kernel,
    out_type=jax.ShapeDtypeStruct((NC, N_PAD, HIDDEN), jnp.float32),
    mesh=_MESH,
    scratch_types=[
        pltpu.VMEM_SHARED((N_PAD, HIDDEN), jnp.float32),
        pltpu.VMEM((NCHW, K), jnp.int32),
        pltpu.VMEM((NCHW, K), jnp.int32),
        pltpu.VMEM((K, HIDDEN), jnp.float32),
    ],
)
def _propagate(p_hbm, src_hbm, dst_hbm, out_hbm, acc_sh, src_v, dst_v,
               rows_f):
    c = lax.axis_index("c")
    s = lax.axis_index("s")

    def zrow(i, carry):
        for jb in range(HIDDEN // 16):
            rows_f[i, pl.ds(jb * 16, 16)] = jnp.zeros((16,), jnp.float32)
        return carry

    lax.fori_loop(0, K, zrow, 0)
    for off, sz in _ROW_CHUNKS:
        pltpu.sync_copy(rows_f.at[pl.ds(0, sz)],
                        acc_sh.at[pl.ds(s * RPT + off, sz)])
    plsc.subcore_barrier()

    def body(j, carry):
        pltpu.sync_copy(p_hbm.at[src_v.at[j]], rows_f)
        pltpu.sync_copy(rows_f, acc_sh.at[dst_v.at[j]], add=True)
        return carry

    for h in range(NWIN):
        pltpu.sync_copy(src_hbm.at[c, s, h], src_v)
        pltpu.sync_copy(dst_hbm.at[c, s, h], dst_v)
        lax.fori_loop(0, NCHW, body, 0)

    plsc.subcore_barrier()
    for off, sz in _ROW_CHUNKS:
        pltpu.sync_copy(acc_sh.at[pl.ds(s * RPT + off, sz)],
                        rows_f.at[pl.ds(0, sz)])
        pltpu.sync_copy(rows_f.at[pl.ds(0, sz)],
                        out_hbm.at[c, pl.ds(s * RPT + off, sz)])


def _entry_body(degp_ref, x_ref, w_ref, dinv_ref, p_ref):
    deg = degp_ref[0, :] + degp_ref[1, :]
    dinv = lax.rsqrt(deg)
    dinv_ref[...] = dinv[:, None]
    p_ref[...] = dinv[:, None] * jnp.dot(
        x_ref[...], w_ref[...], preferred_element_type=jnp.float32)


_tc_entry = pl.pallas_call(
    _entry_body,
    out_shape=(
        jax.ShapeDtypeStruct((N_PAD, 1), jnp.float32),
        jax.ShapeDtypeStruct((N_PAD, IN_DIM), jnp.float32),
    ),
)


def _mid_body(s_ref, dinv_ref, b_ref, w_ref, o_ref):
    h = jnp.maximum(dinv_ref[...] * (s_ref[0] + s_ref[1]) + b_ref[...], 0.0)
    o_ref[...] = dinv_ref[...] * jnp.dot(
        h, w_ref[...], preferred_element_type=jnp.float32)


_tc_mid = pl.pallas_call(
    _mid_body,
    out_shape=jax.ShapeDtypeStruct((N_PAD, HIDDEN), jnp.float32),
)


def _final_body(s_ref, dinv_ref, b_ref, o_ref):
    o_ref[...] = (dinv_ref[...] * (s_ref[0, :, :OUT] + s_ref[1, :, :OUT])
                  + b_ref[...])


_tc_final = pl.pallas_call(
    _final_body,
    out_shape=jax.ShapeDtypeStruct((N_PAD, OUT), jnp.float32),
)


def kernel(x, ei, W1, b1, W2, b2, W3, b3):
    half = E // 2
    nh = N // 2
    loop = jnp.arange(N, dtype=jnp.int32)
    njunk = EPC - half - nh
    junk_src = jnp.full((njunk,), N, dtype=jnp.int32)
    junk_dst = N + jnp.arange(njunk, dtype=jnp.int32) % (N_PAD - N)

    def tiles(v, junk):
        v = v.astype(jnp.int32)
        sc0 = jnp.concatenate([v[:half], loop[:nh], junk])
        sc1 = jnp.concatenate([v[half:], loop[nh:], junk])
        return jnp.stack([sc0, sc1]).reshape(NC, NS, NWIN, NCHW, K)

    src_t = tiles(ei[0], junk_src)
    dst_t = tiles(ei[1], junk_dst)
    xp = jnp.pad(x, ((0, N_PAD - N), (0, 0)))
    init = jnp.zeros((NC * N_PAD,), jnp.float32)
    w1p = W1
    w2p = W2
    w3p = jnp.pad(W3, ((0, 0), (0, HIDDEN - OUT)))

    degp = _degree_kernel(dst_t, init).reshape(NC, N_PAD)
    dinv, p1 = _tc_entry(degp, xp, w1p)
    s1 = _propagate(p1, src_t, dst_t)
    p2 = _tc_mid(s1, dinv, b1.reshape(1, -1), w2p)
    s2 = _propagate(p2, src_t, dst_t)
    p3 = _tc_mid(s2, dinv, b2.reshape(1, -1), w3p)
    s3 = _propagate(p3, src_t, dst_t)
    out = _tc_final(s3, dinv, b3.reshape(1, -1))
    return out[:N]
